# Initial kernel scaffold; baseline (speedup 1.0000x reference)
import jax
import jax.numpy as jnp
from jax.experimental import pallas as pl


def kernel(x_nchw, weight, bias):
    raise NotImplementedError("write your pallas kernel here")


# speedup vs baseline: 1.1265x; 1.1265x over previous
import functools

import jax
import jax.numpy as jnp
from jax.experimental import pallas as pl
from jax.experimental.pallas import tpu as pltpu


def _conv_im2col_kernel(w_ref, b_ref, x_ref, o_ref, *, kh, kw, w_p, l_out, lx):
    xb = x_ref[...].astype(jnp.bfloat16)
    c_in = xb.shape[0]
    taps = []
    for i in range(kh):
        for j in range(kw):
            d = i * w_p + j
            if d + l_out <= lx:
                taps.append(xb[:, d:d + l_out])
            else:
                pad = d + l_out - lx
                taps.append(jnp.concatenate(
                    [xb[:, d:lx],
                     jnp.zeros((c_in, pad), dtype=jnp.bfloat16)], axis=1))
    patches = jnp.concatenate(taps, axis=0)
    acc = jnp.dot(w_ref[...], patches, preferred_element_type=jnp.float32)
    o_ref[...] = acc + b_ref[...]


def kernel(x_nchw, weight, bias):
    n, c_in, h, w = x_nchw.shape
    c_out, c_in_w, kh, kw = weight.shape
    assert c_in == c_in_w
    h_out = h - kh + 1
    w_out = w - kw + 1
    k_dim = kh * kw * c_in
    w_p = w
    l_out = h_out * w_p
    lx = h * w

    w_mat = (jnp.transpose(weight, (0, 2, 3, 1))
             .reshape(c_out, k_dim).astype(jnp.bfloat16))
    b_col = bias.reshape(c_out, 1).astype(jnp.float32)
    x_flat = x_nchw.reshape(n, c_in, lx)

    body = functools.partial(_conv_im2col_kernel,
                             kh=kh, kw=kw, w_p=w_p, l_out=l_out, lx=lx)

    out_flat = pl.pallas_call(
        body,
        out_shape=jax.ShapeDtypeStruct((n, c_out, l_out), jnp.float32),
        grid_spec=pltpu.PrefetchScalarGridSpec(
            num_scalar_prefetch=0,
            grid=(n,),
            in_specs=[
                pl.BlockSpec((c_out, k_dim), lambda b: (0, 0)),
                pl.BlockSpec((c_out, 1), lambda b: (0, 0)),
                pl.BlockSpec((None, c_in, lx), lambda b: (b, 0, 0)),
            ],
            out_specs=pl.BlockSpec((None, c_out, l_out), lambda b: (b, 0, 0)),
        ),
        compiler_params=pltpu.CompilerParams(
            dimension_semantics=("parallel",)),
        cost_estimate=pl.CostEstimate(
            flops=2 * n * c_out * k_dim * l_out,
            transcendentals=0,
            bytes_accessed=4 * (x_flat.size + n * c_out * l_out)
            + 2 * w_mat.size),
    )(w_mat, b_col, x_flat)

    return out_flat.reshape(n, c_out, h_out, w_p)[:, :, :, :w_out]


# speedup vs baseline: 1.2049x; 1.0695x over previous
import functools

import jax
import jax.numpy as jnp
from jax.experimental import pallas as pl
from jax.experimental.pallas import tpu as pltpu


def _conv_im2col_kernel(w_ref, b_ref, x_ref, o_ref, p_ref,
                        *, kh, kw, w_p, l_out, lx, w_out):
    nb, c_in, _ = x_ref.shape
    h_out = l_out // w_p
    def build(img):
        xb = x_ref[img].astype(jnp.bfloat16)
        taps = []
        for i in range(kh):
            for j in range(kw):
                d = i * w_p + j
                if d + l_out <= lx:
                    taps.append(xb[:, d:d + l_out])
                else:
                    pad = d + l_out - lx
                    taps.append(jnp.concatenate(
                        [xb[:, d:lx],
                         jnp.zeros((c_in, pad), dtype=jnp.bfloat16)], axis=1))
        p_ref[img] = jnp.concatenate(taps, axis=0)

    def consume(img):
        acc = jnp.dot(w_ref[...], p_ref[img],
                      preferred_element_type=jnp.float32)
        acc = acc + b_ref[...]
        dense = jnp.concatenate(
            [acc[:, hh * w_p:hh * w_p + w_out] for hh in range(h_out)], axis=1)
        o_ref[img] = dense.reshape(dense.shape[0], h_out, w_out)

    build(0)
    for img in range(nb):
        if img + 1 < nb:
            build(img + 1)
        consume(img)


def kernel(x_nchw, weight, bias):
    n, c_in, h, w = x_nchw.shape
    c_out, c_in_w, kh, kw = weight.shape
    assert c_in == c_in_w
    h_out = h - kh + 1
    w_out = w - kw + 1
    k_dim = kh * kw * c_in
    w_p = w
    l_out = h_out * w_p
    lx = h * w

    w_mat = (jnp.transpose(weight, (0, 2, 3, 1))
             .reshape(c_out, k_dim).astype(jnp.bfloat16))
    b_col = bias.reshape(c_out, 1).astype(jnp.float32)
    x_flat = x_nchw.reshape(n, c_in, lx)

    body = functools.partial(_conv_im2col_kernel,
                             kh=kh, kw=kw, w_p=w_p, l_out=l_out, lx=lx,
                             w_out=w_out)

    m_out = h_out * w_out
    nb = 8
    while n % nb:
        nb //= 2
    out_flat = pl.pallas_call(
        body,
        out_shape=jax.ShapeDtypeStruct((n, c_out, h_out, w_out), jnp.float32),
        grid_spec=pltpu.PrefetchScalarGridSpec(
            num_scalar_prefetch=0,
            grid=(n // nb,),
            in_specs=[
                pl.BlockSpec((c_out, k_dim), lambda b: (0, 0)),
                pl.BlockSpec((c_out, 1), lambda b: (0, 0)),
                pl.BlockSpec((nb, c_in, lx), lambda b: (b, 0, 0)),
            ],
            out_specs=pl.BlockSpec((nb, c_out, h_out, w_out),
                                   lambda b: (b, 0, 0, 0)),
            scratch_shapes=[
                pltpu.VMEM((nb, k_dim, l_out), jnp.bfloat16),
            ],
        ),
        compiler_params=pltpu.CompilerParams(
            dimension_semantics=("parallel",)),
        cost_estimate=pl.CostEstimate(
            flops=2 * n * c_out * k_dim * l_out,
            transcendentals=0,
            bytes_accessed=4 * (x_flat.size + n * c_out * m_out)
            + 2 * w_mat.size),
    )(w_mat, b_col, x_flat)

    return out_flat
